# baseline (speedup 1.0000x reference)
import functools

import jax
import jax.numpy as jnp
from jax import lax
from jax.experimental import pallas as pl
from jax.experimental.pallas import tpu as pltpu
from jax.experimental.pallas import tpu_sc as plsc

_TOKENS = 16384
_D = 2048
_E = 64
_K = 8
_BT = 512
_C = 4
_TCHUNK = _TOKENS // _C

_NC = 2
_NS = 16
_NW = _NC * _NS
_L = 16


def _logits_body(x_ref, w_ref, b_ref, logits_ref, imp_ref):
    i = pl.program_id(0)
    logits = jnp.dot(x_ref[...], w_ref[...], preferred_element_type=jnp.float32)
    logits = logits + b_ref[...]
    logits_ref[:, 0:_E] = logits
    m = jnp.max(logits, axis=1, keepdims=True)
    e = jnp.exp(logits - m)
    p = e / jnp.sum(e, axis=1, keepdims=True)

    @pl.when(i == 0)
    def _():
        imp_ref[...] = jnp.zeros_like(imp_ref)

    imp_ref[...] += jnp.sum(p, axis=0, keepdims=True)


def _compute_logits(x, w, b, chunk):
    steps = _TCHUNK // _BT
    return pl.pallas_call(
        _logits_body,
        grid=(steps,),
        in_specs=[
            pl.BlockSpec((_BT, _D), lambda i, c=chunk, s=steps: (c * s + i, 0)),
            pl.BlockSpec((_D, _E), lambda i: (0, 0)),
            pl.BlockSpec((1, _E), lambda i: (0, 0)),
        ],
        out_specs=[
            pl.BlockSpec((_BT, 128), lambda i: (i, 0)),
            pl.BlockSpec((1, _E), lambda i: (0, 0)),
        ],
        out_shape=[
            jax.ShapeDtypeStruct((_TCHUNK, 128), jnp.float32),
            jax.ShapeDtypeStruct((1, _E), jnp.float32),
        ],
        name=f"gate_logits_c{chunk}",
    )(x, w, b.reshape(1, _E))


def _merge_top16(ka, va, kb, vb):
    krb = lax.rev(kb, (0,))
    vrb = lax.rev(vb, (0,))
    take_a = ka >= krb
    return jnp.where(take_a, ka, krb), jnp.where(take_a, va, vrb)


def _route_body(tpw, logits_hbm, gates_hbm, idx_hbm, load_hbm,
                lbuf, gbuf, ibuf, loadv):
    wid = lax.axis_index("s") * _NC + lax.axis_index("c")
    base = wid * tpw
    pltpu.sync_copy(logits_hbm.at[pl.ds(base, tpw), pl.ds(0, _E)], lbuf)

    lane = lax.iota(jnp.int32, _L)
    mask8 = lane < _K
    ones = jnp.ones((_L,), jnp.float32)
    zeros_i = jnp.zeros((_L,), jnp.int32)
    for j in range(8):
        loadv[0, pl.ds(j * _L, _L)] = jnp.zeros((_L,), jnp.float32)

    @plsc.parallel_loop(0, tpw, 1, unroll=4)
    def _loop(t):
        ks, vs = [], []
        for j in range(4):
            kj, vj = plsc.sort_key_val(
                lbuf[t, pl.ds(j * _L, _L)], lane + j * _L, descending=True)
            ks.append(kj)
            vs.append(vj)
        k01, v01 = _merge_top16(ks[0], vs[0], ks[1], vs[1])
        k01, v01 = plsc.sort_key_val(k01, v01, descending=True)
        k23, v23 = _merge_top16(ks[2], vs[2], ks[3], vs[3])
        k23, v23 = plsc.sort_key_val(k23, v23, descending=True)
        kf, vf = _merge_top16(k01, v01, k23, v23)
        kf, vf = plsc.sort_key_val(kf, vf, descending=True)

        m = jnp.max(kf)
        e = jnp.where(mask8, jnp.exp(kf - m), 0.0)
        g = e / jnp.sum(e)
        plsc.store_compressed(gbuf.at[pl.ds(t * _K, _L)], g, mask=mask8)
        plsc.store_compressed(ibuf.at[pl.ds(t * _K, _L)], vf, mask=mask8)
        plsc.addupdate_scatter(loadv, [zeros_i, vf], ones, mask=mask8)

    pltpu.sync_copy(gbuf.at[pl.ds(0, tpw * _K)],
                    gates_hbm.at[pl.ds(base * _K, tpw * _K)])
    pltpu.sync_copy(ibuf.at[pl.ds(0, tpw * _K)],
                    idx_hbm.at[pl.ds(base * _K, tpw * _K)])
    pltpu.sync_copy(loadv, load_hbm.at[pl.ds(wid, 1)])


@functools.lru_cache(maxsize=None)
def _make_route(tokens):
    tpw = tokens // _NW
    mesh = plsc.VectorSubcoreMesh(
        core_axis_name="c", subcore_axis_name="s",
        num_cores=_NC, num_subcores=_NS)
    return pl.kernel(
        functools.partial(_route_body, tpw),
        out_type=[
            jax.ShapeDtypeStruct((tokens * _K,), jnp.float32),
            jax.ShapeDtypeStruct((tokens * _K,), jnp.int32),
            jax.ShapeDtypeStruct((_NW, 128), jnp.float32),
        ],
        mesh=mesh,
        scratch_types=[
            pltpu.VMEM((tpw, _E), jnp.float32),
            pltpu.VMEM((tpw * _K + _L,), jnp.float32),
            pltpu.VMEM((tpw * _K + _L,), jnp.int32),
            pltpu.VMEM((1, 128), jnp.float32),
        ],
        compiler_params=pltpu.CompilerParams(
            needs_layout_passes=False, use_tc_tiling_on_sc=False),
    )


def _aux_body(*refs):
    imp_refs = refs[:_C]
    load_refs = refs[_C:2 * _C]
    o_ref = refs[2 * _C]
    imp = imp_refs[0][...]
    for r in imp_refs[1:]:
        imp = imp + r[...]
    load = jnp.sum(load_refs[0][:, 0:_E], axis=0, keepdims=True)
    for r in load_refs[1:]:
        load = load + jnp.sum(r[:, 0:_E], axis=0, keepdims=True)
    o_ref[...] = jnp.sum(imp * load, axis=(0, 1), keepdims=True) * float(_E)


def _aux_finish(imps, loads):
    return pl.pallas_call(
        _aux_body,
        out_shape=jax.ShapeDtypeStruct((1, 1), jnp.float32),
        name="aux_finish",
    )(*imps, *loads)


def kernel(x, W_gate, b_gate):
    imps, loads, gates, idxs = [], [], [], []
    route = _make_route(_TCHUNK)
    for c in range(_C):
        logits_c, imp_c = _compute_logits(x, W_gate, b_gate, c)
        g_c, i_c, load_c = route(logits_c)
        imps.append(imp_c)
        loads.append(load_c)
        gates.append(g_c)
        idxs.append(i_c)
    aux = _aux_finish(imps, loads)
    return (jnp.concatenate(gates).reshape(_TOKENS, _K),
            jnp.concatenate(idxs).reshape(_TOKENS, _K),
            aux.reshape(()))


# speedup vs baseline: 1.1670x; 1.1670x over previous
import functools

import jax
import jax.numpy as jnp
from jax import lax
from jax.experimental import pallas as pl
from jax.experimental.pallas import tpu as pltpu
from jax.experimental.pallas import tpu_sc as plsc

_TOKENS = 16384
_D = 2048
_E = 64
_K = 8
_BT = 1024
_C = 2
_TCHUNK = _TOKENS // _C

_NC = 2
_NS = 16
_NW = _NC * _NS
_L = 16


def _logits_body(x_ref, w_ref, b_ref, logits_ref, imp_ref):
    i = pl.program_id(0)
    logits = jnp.dot(x_ref[...], w_ref[...], preferred_element_type=jnp.float32)
    logits = logits + b_ref[...]
    logits_ref[:, 0:_E] = logits
    m = jnp.max(logits, axis=1, keepdims=True)
    e = jnp.exp(logits - m)
    p = e / jnp.sum(e, axis=1, keepdims=True)

    @pl.when(i == 0)
    def _():
        imp_ref[...] = jnp.zeros_like(imp_ref)

    imp_ref[...] += jnp.sum(p, axis=0, keepdims=True)


def _compute_logits(x, w, b, chunk):
    steps = _TCHUNK // _BT
    return pl.pallas_call(
        _logits_body,
        grid=(steps,),
        in_specs=[
            pl.BlockSpec((_BT, _D), lambda i, c=chunk, s=steps: (c * s + i, 0)),
            pl.BlockSpec((_D, _E), lambda i: (0, 0)),
            pl.BlockSpec((1, _E), lambda i: (0, 0)),
        ],
        out_specs=[
            pl.BlockSpec((_BT, 128), lambda i: (i, 0)),
            pl.BlockSpec((1, _E), lambda i: (0, 0)),
        ],
        out_shape=[
            jax.ShapeDtypeStruct((_TCHUNK, 128), jnp.float32),
            jax.ShapeDtypeStruct((1, _E), jnp.float32),
        ],
        name=f"gate_logits_c{chunk}",
    )(x, w, b.reshape(1, _E))


def _merge_top16(ka, va, kb, vb):
    krb = lax.rev(kb, (0,))
    vrb = lax.rev(vb, (0,))
    take_a = ka >= krb
    return jnp.where(take_a, ka, krb), jnp.where(take_a, va, vrb)


def _route_body(tpw, logits_hbm, gates_hbm, idx_hbm, load_hbm,
                lbuf, gbuf, ibuf, loadv):
    wid = lax.axis_index("s") * _NC + lax.axis_index("c")
    base = wid * tpw
    pltpu.sync_copy(logits_hbm.at[pl.ds(base, tpw), pl.ds(0, _E)], lbuf)

    lane = lax.iota(jnp.int32, _L)
    mask8 = lane < _K
    ones = jnp.ones((_L,), jnp.float32)
    zeros_i = jnp.zeros((_L,), jnp.int32)
    for j in range(8):
        loadv[0, pl.ds(j * _L, _L)] = jnp.zeros((_L,), jnp.float32)

    @plsc.parallel_loop(0, tpw, 1, unroll=4)
    def _loop(t):
        ks, vs = [], []
        for j in range(4):
            kj, vj = plsc.sort_key_val(
                lbuf[t, pl.ds(j * _L, _L)], lane + j * _L, descending=True)
            ks.append(kj)
            vs.append(vj)
        k01, v01 = _merge_top16(ks[0], vs[0], ks[1], vs[1])
        k01, v01 = plsc.sort_key_val(k01, v01, descending=True)
        k23, v23 = _merge_top16(ks[2], vs[2], ks[3], vs[3])
        k23, v23 = plsc.sort_key_val(k23, v23, descending=True)
        kf, vf = _merge_top16(k01, v01, k23, v23)
        kf, vf = plsc.sort_key_val(kf, vf, descending=True)

        m = jnp.max(kf)
        e = jnp.where(mask8, jnp.exp(kf - m), 0.0)
        g = e / jnp.sum(e)
        plsc.store_compressed(gbuf.at[pl.ds(t * _K, _L)], g, mask=mask8)
        plsc.store_compressed(ibuf.at[pl.ds(t * _K, _L)], vf, mask=mask8)
        plsc.addupdate_scatter(loadv, [zeros_i, vf], ones, mask=mask8)

    pltpu.sync_copy(gbuf.at[pl.ds(0, tpw * _K)],
                    gates_hbm.at[pl.ds(base * _K, tpw * _K)])
    pltpu.sync_copy(ibuf.at[pl.ds(0, tpw * _K)],
                    idx_hbm.at[pl.ds(base * _K, tpw * _K)])
    pltpu.sync_copy(loadv, load_hbm.at[pl.ds(wid, 1)])


@functools.lru_cache(maxsize=None)
def _make_route(tokens):
    tpw = tokens // _NW
    mesh = plsc.VectorSubcoreMesh(
        core_axis_name="c", subcore_axis_name="s",
        num_cores=_NC, num_subcores=_NS)
    return pl.kernel(
        functools.partial(_route_body, tpw),
        out_type=[
            jax.ShapeDtypeStruct((tokens * _K,), jnp.float32),
            jax.ShapeDtypeStruct((tokens * _K,), jnp.int32),
            jax.ShapeDtypeStruct((_NW, 128), jnp.float32),
        ],
        mesh=mesh,
        scratch_types=[
            pltpu.VMEM((tpw, _E), jnp.float32),
            pltpu.VMEM((tpw * _K + _L,), jnp.float32),
            pltpu.VMEM((tpw * _K + _L,), jnp.int32),
            pltpu.VMEM((1, 128), jnp.float32),
        ],
        compiler_params=pltpu.CompilerParams(
            needs_layout_passes=False, use_tc_tiling_on_sc=False),
    )


def _aux_body(*refs):
    imp_refs = refs[:_C]
    load_refs = refs[_C:2 * _C]
    o_ref = refs[2 * _C]
    imp = imp_refs[0][...]
    for r in imp_refs[1:]:
        imp = imp + r[...]
    load = jnp.sum(load_refs[0][:, 0:_E], axis=0, keepdims=True)
    for r in load_refs[1:]:
        load = load + jnp.sum(r[:, 0:_E], axis=0, keepdims=True)
    o_ref[...] = jnp.sum(imp * load, axis=(0, 1), keepdims=True) * float(_E)


def _aux_finish(imps, loads):
    return pl.pallas_call(
        _aux_body,
        out_shape=jax.ShapeDtypeStruct((1, 1), jnp.float32),
        name="aux_finish",
    )(*imps, *loads)


def kernel(x, W_gate, b_gate):
    imps, loads, gates, idxs = [], [], [], []
    route = _make_route(_TCHUNK)
    for c in range(_C):
        logits_c, imp_c = _compute_logits(x, W_gate, b_gate, c)
        g_c, i_c, load_c = route(logits_c)
        imps.append(imp_c)
        loads.append(load_c)
        gates.append(g_c)
        idxs.append(i_c)
    aux = _aux_finish(imps, loads)
    return (jnp.concatenate(gates).reshape(_TOKENS, _K),
            jnp.concatenate(idxs).reshape(_TOKENS, _K),
            aux.reshape(()))
